# baseline (speedup 1.0000x reference)
import functools

import jax
import jax.numpy as jnp
from jax import lax
from jax.experimental import pallas as pl
from jax.experimental.pallas import tpu as pltpu

N = 4096
D = 512
K = 32
RB = 256
NCH = 8
CW = N // NCH
L = 16
NEG_INF = float("-inf")


def _normalize_body(x_ref, out_ref):
    x = x_ref[...]
    sq = x * x
    p = sq[:, 0:128] + sq[:, 128:256]
    p = p + sq[:, 256:384]
    p = p + sq[:, 384:512]
    r3 = p.reshape(p.shape[0], 16, 8)
    acc = r3[:, 0, :]
    for j in range(1, 16):
        acc = acc + r3[:, j, :]
    t = acc[:, 0:4] + acc[:, 4:8]
    t = t[:, 0:2] + t[:, 2:4]
    n2 = t[:, 0:1] + t[:, 1:2]
    s = n2 * lax.rsqrt(n2)
    s = jnp.where(n2 == 0.0, 0.0, s)
    norm = jnp.maximum(s, 1e-12)
    out_ref[...] = x * (1.0 / norm)


def _topk_body(x_rows_ref, x_all_ref, adj_ref, vals_ref, idx_ref,
               cand_ref, work_ref):
    x = x_rows_ref[...]
    x_all = x_all_ref[...]
    adj = adj_ref[...]
    sim = lax.dot_general(
        x.astype(jnp.bfloat16), x_all.astype(jnp.bfloat16),
        (((1,), (1,)), ((), ())),
        preferred_element_type=jnp.float32,
    )
    mask = adj > 0.0
    w = jnp.where(mask, sim * adj, 0.0)
    rs = jnp.sum(w, axis=1, keepdims=True)
    probs = w / rs
    cand = jnp.where(mask, probs, NEG_INF)
    cand_ref[...] = cand
    work_ref[...] = cand

    col = lax.broadcasted_iota(jnp.int32, (RB, N), 1)
    kcolL = lax.broadcasted_iota(jnp.int32, (RB, L), 1)
    kcolK = lax.broadcasted_iota(jnp.int32, (RB, K), 1)

    cand_v = []
    cand_i = []
    last_v = []
    for c in range(NCH):
        sl = pl.ds(c * CW, CW)
        colc = col[:, c * CW:(c + 1) * CW]

        def cstep(t, carry, sl=sl, colc=colc):
            lv, li = carry
            ch = work_ref[:, sl]
            m = jnp.max(ch, axis=1, keepdims=True)
            sel = jnp.min(jnp.where(ch == m, colc, N), axis=1, keepdims=True)
            work_ref[:, sl] = jnp.where(colc == sel, NEG_INF, ch)
            lv = jnp.where(kcolL == t, m, lv)
            li = jnp.where(kcolL == t, sel, li)
            return lv, li

        lv, li = lax.fori_loop(
            0, L, cstep,
            (jnp.zeros((RB, L), jnp.float32), jnp.zeros((RB, L), jnp.int32)))
        cand_v.append(lv)
        cand_i.append(li)
        last_v.append(lv[:, L - 1:L])

    V = jnp.concatenate(cand_v, axis=1)
    I = jnp.concatenate(cand_i, axis=1)
    v_last = jnp.concatenate(last_v, axis=1)

    P = NCH * L
    pos = lax.broadcasted_iota(jnp.int32, (RB, P), 1)

    def mstep(t, carry):
        Vc, vals, idxs = carry
        m = jnp.max(Vc, axis=1, keepdims=True)
        selp = jnp.min(jnp.where(Vc == m, pos, P), axis=1, keepdims=True)
        hit = pos == selp
        selidx = jnp.min(jnp.where(hit, I, N), axis=1, keepdims=True)
        Vc = jnp.where(hit, NEG_INF, Vc)
        vals = jnp.where(kcolK == t, m, vals)
        idxs = jnp.where(kcolK == t, selidx, idxs)
        return Vc, vals, idxs

    _, vals_f, idxs_f = lax.fori_loop(
        0, K, mstep,
        (V, jnp.zeros((RB, K), jnp.float32), jnp.zeros((RB, K), jnp.int32)))

    tau = vals_f[:, K - 1:K]
    saturated = jnp.any(v_last >= tau)

    def full_extraction(_):
        def step(t, carry):
            vals, idxs = carry
            c2 = cand_ref[...]
            m = jnp.max(c2, axis=1, keepdims=True)
            sel = jnp.min(jnp.where(c2 == m, col, N), axis=1, keepdims=True)
            cand_ref[...] = jnp.where(col == sel, NEG_INF, c2)
            vals = jnp.where(kcolK == t, m, vals)
            idxs = jnp.where(kcolK == t, sel, idxs)
            return vals, idxs
        return lax.fori_loop(
            0, K, step,
            (jnp.zeros((RB, K), jnp.float32), jnp.zeros((RB, K), jnp.int32)))

    vals_o, idxs_o = lax.cond(
        saturated, full_extraction, lambda _: (vals_f, idxs_f), operand=None)
    vals_ref[...] = vals_o
    idx_ref[...] = idxs_o


def kernel(adjacency_matrix, transaction_record, labels):
    del labels
    x_norm = pl.pallas_call(
        _normalize_body,
        grid=(N // 512,),
        in_specs=[pl.BlockSpec((512, D), lambda i: (i, 0))],
        out_specs=pl.BlockSpec((512, D), lambda i: (i, 0)),
        out_shape=jax.ShapeDtypeStruct((N, D), jnp.float32),
    )(transaction_record)

    vals, idxs = pl.pallas_call(
        _topk_body,
        grid=(N // RB,),
        in_specs=[
            pl.BlockSpec((RB, D), lambda i: (i, 0)),
            pl.BlockSpec((N, D), lambda i: (0, 0)),
            pl.BlockSpec((RB, N), lambda i: (i, 0)),
        ],
        out_specs=[
            pl.BlockSpec((RB, K), lambda i: (i, 0)),
            pl.BlockSpec((RB, K), lambda i: (i, 0)),
        ],
        out_shape=[
            jax.ShapeDtypeStruct((N, K), jnp.float32),
            jax.ShapeDtypeStruct((N, K), jnp.int32),
        ],
        scratch_shapes=[pltpu.VMEM((RB, N), jnp.float32),
                        pltpu.VMEM((RB, N), jnp.float32)],
    )(x_norm, x_norm, adjacency_matrix)
    return vals, idxs


# speedup vs baseline: 1.0017x; 1.0017x over previous
import functools

import jax
import jax.numpy as jnp
from jax import lax
from jax.experimental import pallas as pl
from jax.experimental.pallas import tpu as pltpu

N = 4096
D = 512
K = 32
RB = 256
NCH = 8
CW = N // NCH
L = 16
NEG_INF = float("-inf")


def _normalize_body(x_ref, out_ref):
    x = x_ref[...]
    sq = x * x
    p = sq[:, 0:128] + sq[:, 128:256]
    p = p + sq[:, 256:384]
    p = p + sq[:, 384:512]
    r3 = p.reshape(p.shape[0], 16, 8)
    acc = r3[:, 0, :]
    for j in range(1, 16):
        acc = acc + r3[:, j, :]
    t = acc[:, 0:4] + acc[:, 4:8]
    t = t[:, 0:2] + t[:, 2:4]
    n2 = t[:, 0:1] + t[:, 1:2]
    s = n2 * lax.rsqrt(n2)
    s = jnp.where(n2 == 0.0, 0.0, s)
    norm = jnp.maximum(s, 1e-12)
    out_ref[...] = x * (1.0 / norm)


def _topk_body(x_rows_ref, x_all_ref, adj_ref, vals_ref, idx_ref,
               cand_ref, work_ref):
    x = x_rows_ref[...]
    x_all = x_all_ref[...]
    adj = adj_ref[...]
    sim = lax.dot_general(
        x.astype(jnp.bfloat16), x_all.astype(jnp.bfloat16),
        (((1,), (1,)), ((), ())),
        preferred_element_type=jnp.float32,
    )
    mask = adj > 0.0
    w = jnp.where(mask, sim * adj, 0.0)
    rs = jnp.sum(w, axis=1, keepdims=True)
    probs = w / rs
    cand = jnp.where(mask, probs, NEG_INF)
    cand_ref[...] = cand
    work_ref[...] = cand

    col = lax.broadcasted_iota(jnp.int32, (RB, N), 1)
    kcolL = lax.broadcasted_iota(jnp.int32, (RB, L), 1)
    kcolK = lax.broadcasted_iota(jnp.int32, (RB, K), 1)

    cand_v = []
    cand_i = []
    last_v = []
    for c in range(NCH):
        sl = pl.ds(c * CW, CW)
        colc = col[:, c * CW:(c + 1) * CW]

        def cstep(t, carry, sl=sl, colc=colc):
            lv, li = carry
            ch = work_ref[:, sl]
            m = jnp.max(ch, axis=1, keepdims=True)
            sel = jnp.min(jnp.where(ch == m, colc, N), axis=1, keepdims=True)
            work_ref[:, sl] = jnp.where(colc == sel, NEG_INF, ch)
            lv = jnp.where(kcolL == t, m, lv)
            li = jnp.where(kcolL == t, sel, li)
            return lv, li

        lv, li = lax.fori_loop(
            0, L, cstep,
            (jnp.zeros((RB, L), jnp.float32), jnp.zeros((RB, L), jnp.int32)))
        cand_v.append(lv)
        cand_i.append(li)
        last_v.append(lv[:, L - 1:L])

    V = jnp.concatenate(cand_v, axis=1)
    I = jnp.concatenate(cand_i, axis=1)
    v_last = jnp.concatenate(last_v, axis=1)

    P = NCH * L
    pos = lax.broadcasted_iota(jnp.int32, (RB, P), 1)

    def mstep(t, carry):
        Vc, vals, idxs = carry
        m = jnp.max(Vc, axis=1, keepdims=True)
        selp = jnp.min(jnp.where(Vc == m, pos, P), axis=1, keepdims=True)
        hit = pos == selp
        selidx = jnp.min(jnp.where(hit, I, N), axis=1, keepdims=True)
        Vc = jnp.where(hit, NEG_INF, Vc)
        vals = jnp.where(kcolK == t, m, vals)
        idxs = jnp.where(kcolK == t, selidx, idxs)
        return Vc, vals, idxs

    _, vals_f, idxs_f = lax.fori_loop(
        0, K, mstep,
        (V, jnp.zeros((RB, K), jnp.float32), jnp.zeros((RB, K), jnp.int32)))

    tau = vals_f[:, K - 1:K]
    saturated = jnp.any(v_last >= tau)

    vals_ref[...] = vals_f
    idx_ref[...] = idxs_f

    @pl.when(saturated)
    def _full_extraction():
        def step(t, carry):
            vals, idxs = carry
            c2 = cand_ref[...]
            m = jnp.max(c2, axis=1, keepdims=True)
            sel = jnp.min(jnp.where(c2 == m, col, N), axis=1, keepdims=True)
            cand_ref[...] = jnp.where(col == sel, NEG_INF, c2)
            vals = jnp.where(kcolK == t, m, vals)
            idxs = jnp.where(kcolK == t, sel, idxs)
            return vals, idxs
        vals_x, idxs_x = lax.fori_loop(
            0, K, step,
            (jnp.zeros((RB, K), jnp.float32), jnp.zeros((RB, K), jnp.int32)))
        vals_ref[...] = vals_x
        idx_ref[...] = idxs_x


def kernel(adjacency_matrix, transaction_record, labels):
    del labels
    x_norm = pl.pallas_call(
        _normalize_body,
        grid=(N // 512,),
        in_specs=[pl.BlockSpec((512, D), lambda i: (i, 0))],
        out_specs=pl.BlockSpec((512, D), lambda i: (i, 0)),
        out_shape=jax.ShapeDtypeStruct((N, D), jnp.float32),
    )(transaction_record)

    vals, idxs = pl.pallas_call(
        _topk_body,
        grid=(N // RB,),
        in_specs=[
            pl.BlockSpec((RB, D), lambda i: (i, 0)),
            pl.BlockSpec((N, D), lambda i: (0, 0)),
            pl.BlockSpec((RB, N), lambda i: (i, 0)),
        ],
        out_specs=[
            pl.BlockSpec((RB, K), lambda i: (i, 0)),
            pl.BlockSpec((RB, K), lambda i: (i, 0)),
        ],
        out_shape=[
            jax.ShapeDtypeStruct((N, K), jnp.float32),
            jax.ShapeDtypeStruct((N, K), jnp.int32),
        ],
        scratch_shapes=[pltpu.VMEM((RB, N), jnp.float32),
                        pltpu.VMEM((RB, N), jnp.float32)],
    )(x_norm, x_norm, adjacency_matrix)
    return vals, idxs


# speedup vs baseline: 1.3827x; 1.3803x over previous
import functools

import jax
import jax.numpy as jnp
from jax import lax
from jax.experimental import pallas as pl
from jax.experimental.pallas import tpu as pltpu

N = 4096
D = 512
K = 32
RB = 256
NCH = 8
CW = N // NCH
L = 16
NEG_INF = float("-inf")


def _normalize_body(x_ref, out_ref):
    x = x_ref[...]
    sq = x * x
    p = sq[:, 0:128] + sq[:, 128:256]
    p = p + sq[:, 256:384]
    p = p + sq[:, 384:512]
    r3 = p.reshape(p.shape[0], 16, 8)
    acc = r3[:, 0, :]
    for j in range(1, 16):
        acc = acc + r3[:, j, :]
    t = acc[:, 0:4] + acc[:, 4:8]
    t = t[:, 0:2] + t[:, 2:4]
    n2 = t[:, 0:1] + t[:, 1:2]
    s = n2 * lax.rsqrt(n2)
    s = jnp.where(n2 == 0.0, 0.0, s)
    norm = jnp.maximum(s, 1e-12)
    out_ref[...] = x * (1.0 / norm)


def _topk_body(x_rows_ref, x_all_ref, adj_ref, vals_ref, idx_ref,
               cand_ref, work_ref):
    x = x_rows_ref[...]
    x_all = x_all_ref[...]
    adj = adj_ref[...]
    sim = lax.dot_general(
        x.astype(jnp.bfloat16), x_all.astype(jnp.bfloat16),
        (((1,), (1,)), ((), ())),
        preferred_element_type=jnp.float32,
    )
    mask = adj > 0.0
    w = jnp.where(mask, sim * adj, 0.0)
    rs = jnp.sum(w, axis=1, keepdims=True)
    probs = w / rs
    cand = jnp.where(mask, probs, NEG_INF)
    cand_ref[...] = cand
    work_ref[...] = cand

    col = lax.broadcasted_iota(jnp.int32, (RB, N), 1)
    kcolK = lax.broadcasted_iota(jnp.int32, (RB, K), 1)
    P = NCH * L
    posP = lax.broadcasted_iota(jnp.int32, (RB, P), 1)

    def cstep(t, carry):
        V, I = carry
        for c in range(NCH):
            sl = pl.ds(c * CW, CW)
            colc = col[:, c * CW:(c + 1) * CW]
            ch = work_ref[:, sl]
            m = jnp.max(ch, axis=1, keepdims=True)
            sel = jnp.min(jnp.where(ch == m, colc, N), axis=1, keepdims=True)
            work_ref[:, sl] = jnp.where(colc == sel, NEG_INF, ch)
            V = jnp.where(posP == c * L + t, m, V)
            I = jnp.where(posP == c * L + t, sel, I)
        return V, I

    V, I = lax.fori_loop(
        0, L, cstep,
        (jnp.full((RB, P), NEG_INF, jnp.float32), jnp.zeros((RB, P), jnp.int32)))
    v_last = jnp.concatenate(
        [V[:, c * L + L - 1:c * L + L] for c in range(NCH)], axis=1)

    def mstep(t, carry):
        Vc, vals, idxs = carry
        m = jnp.max(Vc, axis=1, keepdims=True)
        ismax = Vc == m
        selidx = jnp.min(jnp.where(ismax, I, N), axis=1, keepdims=True)
        Vc = jnp.where(ismax & (I == selidx), NEG_INF, Vc)
        vals = jnp.where(kcolK == t, m, vals)
        idxs = jnp.where(kcolK == t, selidx, idxs)
        return Vc, vals, idxs

    _, vals_f, idxs_f = lax.fori_loop(
        0, K, mstep,
        (V, jnp.zeros((RB, K), jnp.float32), jnp.zeros((RB, K), jnp.int32)))

    tau = vals_f[:, K - 1:K]
    saturated = jnp.any(v_last >= tau)

    vals_ref[...] = vals_f
    idx_ref[...] = idxs_f

    @pl.when(saturated)
    def _full_extraction():
        def step(t, carry):
            vals, idxs = carry
            c2 = cand_ref[...]
            m = jnp.max(c2, axis=1, keepdims=True)
            sel = jnp.min(jnp.where(c2 == m, col, N), axis=1, keepdims=True)
            cand_ref[...] = jnp.where(col == sel, NEG_INF, c2)
            vals = jnp.where(kcolK == t, m, vals)
            idxs = jnp.where(kcolK == t, sel, idxs)
            return vals, idxs
        vals_x, idxs_x = lax.fori_loop(
            0, K, step,
            (jnp.zeros((RB, K), jnp.float32), jnp.zeros((RB, K), jnp.int32)))
        vals_ref[...] = vals_x
        idx_ref[...] = idxs_x


def kernel(adjacency_matrix, transaction_record, labels):
    del labels
    x_norm = pl.pallas_call(
        _normalize_body,
        grid=(N // 512,),
        in_specs=[pl.BlockSpec((512, D), lambda i: (i, 0))],
        out_specs=pl.BlockSpec((512, D), lambda i: (i, 0)),
        out_shape=jax.ShapeDtypeStruct((N, D), jnp.float32),
    )(transaction_record)

    vals, idxs = pl.pallas_call(
        _topk_body,
        grid=(N // RB,),
        in_specs=[
            pl.BlockSpec((RB, D), lambda i: (i, 0)),
            pl.BlockSpec((N, D), lambda i: (0, 0)),
            pl.BlockSpec((RB, N), lambda i: (i, 0)),
        ],
        out_specs=[
            pl.BlockSpec((RB, K), lambda i: (i, 0)),
            pl.BlockSpec((RB, K), lambda i: (i, 0)),
        ],
        out_shape=[
            jax.ShapeDtypeStruct((N, K), jnp.float32),
            jax.ShapeDtypeStruct((N, K), jnp.int32),
        ],
        scratch_shapes=[pltpu.VMEM((RB, N), jnp.float32),
                        pltpu.VMEM((RB, N), jnp.float32)],
    )(x_norm, x_norm, adjacency_matrix)
    return vals, idxs


# speedup vs baseline: 1.4907x; 1.0781x over previous
import functools

import jax
import jax.numpy as jnp
from jax import lax
from jax.experimental import pallas as pl
from jax.experimental.pallas import tpu as pltpu

N = 4096
D = 512
K = 32
RB = 256
NEG_INF = float("-inf")


def _normalize_body(x_ref, out_ref):
    x = x_ref[...]
    sq = x * x
    p = sq[:, 0:128] + sq[:, 128:256]
    p = p + sq[:, 256:384]
    p = p + sq[:, 384:512]
    acc = p
    for j in range(1, 16):
        acc = acc + pltpu.roll(p, 128 - 8 * j, 1)
    t = acc + pltpu.roll(acc, 124, 1)
    t = t + pltpu.roll(t, 126, 1)
    t = t + pltpu.roll(t, 127, 1)
    n2 = t[:, 0:1]
    s = n2 * lax.rsqrt(n2)
    s = jnp.where(n2 == 0.0, 0.0, s)
    norm = jnp.maximum(s, 1e-12)
    out_ref[...] = x * (1.0 / norm)


def _topk_body(x_rows_ref, x_all_ref, adj_ref, vals_ref, idx_ref, cand_ref):
    x = x_rows_ref[...]
    x_all = x_all_ref[...]
    adj = adj_ref[...]
    sim = lax.dot_general(
        x.astype(jnp.bfloat16), x_all.astype(jnp.bfloat16),
        (((1,), (1,)), ((), ())),
        preferred_element_type=jnp.float32,
    )
    mask = adj > 0.0
    w = jnp.where(mask, sim * adj, 0.0)
    rs = jnp.sum(w, axis=1, keepdims=True)
    probs = w / rs
    cand_ref[...] = jnp.where(mask, probs, NEG_INF)

    col = lax.broadcasted_iota(jnp.int32, (RB, N), 1)
    kcol = lax.broadcasted_iota(jnp.int32, (RB, K), 1)

    def step(t, carry):
        vals, idxs = carry
        c = cand_ref[...]
        m = jnp.max(c, axis=1, keepdims=True)
        sel = jnp.min(jnp.where(c == m, col, N), axis=1, keepdims=True)
        cand_ref[...] = jnp.where(col == sel, NEG_INF, c)
        vals = jnp.where(kcol == t, m, vals)
        idxs = jnp.where(kcol == t, sel, idxs)
        return vals, idxs

    vals0 = jnp.zeros((RB, K), jnp.float32)
    idxs0 = jnp.zeros((RB, K), jnp.int32)
    vals, idxs = lax.fori_loop(0, K, step, (vals0, idxs0))
    vals_ref[...] = vals
    idx_ref[...] = idxs


def kernel(adjacency_matrix, transaction_record, labels):
    del labels
    x_norm = pl.pallas_call(
        _normalize_body,
        grid=(N // 512,),
        in_specs=[pl.BlockSpec((512, D), lambda i: (i, 0))],
        out_specs=pl.BlockSpec((512, D), lambda i: (i, 0)),
        out_shape=jax.ShapeDtypeStruct((N, D), jnp.float32),
    )(transaction_record)

    vals, idxs = pl.pallas_call(
        _topk_body,
        grid=(N // RB,),
        in_specs=[
            pl.BlockSpec((RB, D), lambda i: (i, 0)),
            pl.BlockSpec((N, D), lambda i: (0, 0)),
            pl.BlockSpec((RB, N), lambda i: (i, 0)),
        ],
        out_specs=[
            pl.BlockSpec((RB, K), lambda i: (i, 0)),
            pl.BlockSpec((RB, K), lambda i: (i, 0)),
        ],
        out_shape=[
            jax.ShapeDtypeStruct((N, K), jnp.float32),
            jax.ShapeDtypeStruct((N, K), jnp.int32),
        ],
        scratch_shapes=[pltpu.VMEM((RB, N), jnp.float32)],
    )(x_norm, x_norm, adjacency_matrix)
    return vals, idxs


# speedup vs baseline: 1.5705x; 1.0536x over previous
import functools

import jax
import jax.numpy as jnp
from jax import lax
from jax.experimental import pallas as pl
from jax.experimental.pallas import tpu as pltpu

N = 4096
D = 512
K = 32
RB = 512
NEG_INF = float("-inf")


def _normalize_body(x_ref, out_ref):
    x = x_ref[...]
    sq = x * x
    p = sq[:, 0:128] + sq[:, 128:256]
    p = p + sq[:, 256:384]
    p = p + sq[:, 384:512]
    acc = p
    for j in range(1, 16):
        acc = acc + pltpu.roll(p, 128 - 8 * j, 1)
    t = acc + pltpu.roll(acc, 124, 1)
    t = t + pltpu.roll(t, 126, 1)
    t = t + pltpu.roll(t, 127, 1)
    n2 = t[:, 0:1]
    s = n2 * lax.rsqrt(n2)
    s = jnp.where(n2 == 0.0, 0.0, s)
    norm = jnp.maximum(s, 1e-12)
    out_ref[...] = x * (1.0 / norm)


def _topk_body(x_rows_ref, x_all_ref, adj_ref, vals_ref, idx_ref, cand_ref):
    x = x_rows_ref[...]
    x_all = x_all_ref[...]
    adj = adj_ref[...]
    sim = lax.dot_general(
        x.astype(jnp.bfloat16), x_all.astype(jnp.bfloat16),
        (((1,), (1,)), ((), ())),
        preferred_element_type=jnp.float32,
    )
    mask = adj > 0.0
    w = jnp.where(mask, sim * adj, 0.0)
    rs = jnp.sum(w, axis=1, keepdims=True)
    probs = w / rs
    cand_ref[...] = jnp.where(mask, probs, NEG_INF)

    col = lax.broadcasted_iota(jnp.int32, (RB, N), 1)
    kcol = lax.broadcasted_iota(jnp.int32, (RB, K), 1)

    def step(t, carry):
        vals, idxs = carry
        c = cand_ref[...]
        m = jnp.max(c, axis=1, keepdims=True)
        sel = jnp.min(jnp.where(c == m, col, N), axis=1, keepdims=True)
        cand_ref[...] = jnp.where(col == sel, NEG_INF, c)
        vals = jnp.where(kcol == t, m, vals)
        idxs = jnp.where(kcol == t, sel, idxs)
        return vals, idxs

    vals0 = jnp.zeros((RB, K), jnp.float32)
    idxs0 = jnp.zeros((RB, K), jnp.int32)
    vals, idxs = lax.fori_loop(0, K, step, (vals0, idxs0))
    vals_ref[...] = vals
    idx_ref[...] = idxs


def kernel(adjacency_matrix, transaction_record, labels):
    del labels
    x_norm = pl.pallas_call(
        _normalize_body,
        grid=(N // 512,),
        in_specs=[pl.BlockSpec((512, D), lambda i: (i, 0))],
        out_specs=pl.BlockSpec((512, D), lambda i: (i, 0)),
        out_shape=jax.ShapeDtypeStruct((N, D), jnp.float32),
    )(transaction_record)

    vals, idxs = pl.pallas_call(
        _topk_body,
        grid=(N // RB,),
        in_specs=[
            pl.BlockSpec((RB, D), lambda i: (i, 0)),
            pl.BlockSpec((N, D), lambda i: (0, 0)),
            pl.BlockSpec((RB, N), lambda i: (i, 0)),
        ],
        out_specs=[
            pl.BlockSpec((RB, K), lambda i: (i, 0)),
            pl.BlockSpec((RB, K), lambda i: (i, 0)),
        ],
        out_shape=[
            jax.ShapeDtypeStruct((N, K), jnp.float32),
            jax.ShapeDtypeStruct((N, K), jnp.int32),
        ],
        scratch_shapes=[pltpu.VMEM((RB, N), jnp.float32)],
    )(x_norm, x_norm, adjacency_matrix)
    return vals, idxs
